# baseline (speedup 1.0000x reference)
import functools

import jax
import jax.numpy as jnp
from jax import lax
from jax.experimental import pallas as pl
from jax.experimental.pallas import tpu as pltpu
from jax.experimental.pallas import tpu_sc as plsc

_B, _H, _W = 4, 256, 256
_NCELL = _B * _H * _W
_D = 128
_NC, _NS, _L = 2, 16, 16
_NWORK = _NC * _NS
_CELLS = 2048
_NPASS = _NCELL // (_NC * _CELLS)
_RPT = _CELLS // _NS
_K = 16
_DUMP = _CELLS


def _sc_segment_sum(feat, flat_idx):
    V = feat.shape[0]
    vpw = V // _NS
    nvec = vpw // _L

    mesh = plsc.VectorSubcoreMesh(core_axis_name="c", subcore_axis_name="s")

    @functools.partial(
        pl.kernel,
        mesh=mesh,
        out_type=(
            jax.ShapeDtypeStruct((_NCELL, _D), jnp.float32),
            jax.ShapeDtypeStruct((_NCELL, _L), jnp.float32),
        ),
        scratch_types=[
            pltpu.VMEM((vpw,), jnp.int32),
            pltpu.VMEM((vpw + _K + _L,), jnp.int32),
            pltpu.VMEM((vpw + _K + _L,), jnp.int32),
            pltpu.VMEM((_K,), jnp.int32),
            pltpu.VMEM((_K,), jnp.int32),
            pltpu.VMEM((_K, _D), jnp.float32),
            pltpu.VMEM((_K, _L), jnp.float32),
            pltpu.VMEM((_RPT // 2, _D), jnp.float32),
            pltpu.VMEM((_RPT, _L), jnp.float32),
            pltpu.VMEM_SHARED((_CELLS + 8, _D), jnp.float32),
            pltpu.VMEM_SHARED((_CELLS + 8, _L), jnp.float32),
            pltpu.SemaphoreType.DMA,
        ],
        compiler_params=pltpu.CompilerParams(needs_layout_passes=False),
    )
    def k(feat_h, idx_h, sums_h, cnts_h, idx_t, pend_vox, pend_cell,
          stage_vox, stage_cell, rows, ones, zrows, zcnt, acc, cnt, gsem):
        cid = lax.axis_index("c")
        sid = lax.axis_index("s")
        z16 = jnp.zeros((_L,), jnp.float32)
        o16 = jnp.ones((_L,), jnp.float32)

        def zrow_body(i, carry):
            for t in range(_D // _L):
                zrows[i, pl.ds(t * _L, _L)] = z16
            return carry

        lax.fori_loop(0, _RPT // 2, zrow_body, 0)

        def zcnt_body(i, carry):
            zcnt[i, :] = z16
            return carry

        lax.fori_loop(0, _RPT, zcnt_body, 0)

        def ones_body(i, carry):
            ones[i, :] = o16
            return carry

        lax.fori_loop(0, _K, ones_body, 0)

        pltpu.sync_copy(idx_h.at[pl.ds(sid * vpw, vpw)], idx_t)

        row0 = sid * _RPT
        half = _RPT // 2
        pltpu.sync_copy(zrows, acc.at[pl.ds(row0, half)])
        pltpu.sync_copy(zrows, acc.at[pl.ds(row0 + half, half)])
        pltpu.sync_copy(zcnt, cnt.at[pl.ds(row0, _RPT)])
        plsc.subcore_barrier()

        voxbase = sid * vpw

        lanes = lax.iota(jnp.int32, _L)
        zero16 = jnp.zeros((_L,), jnp.int32)
        cells16 = jnp.full((_L,), _CELLS, jnp.int32)
        dump16 = jnp.full((_L,), _DUMP, jnp.int32)
        trash16 = jnp.full((_L,), vpw + _K + _L - 1, jnp.int32)

        def pass_body(p, carry):
            base = p * (_NC * _CELLS) + cid * _CELLS
            base16 = jnp.full((_L,), base, jnp.int32)
            vox0 = jnp.full((_L,), voxbase, jnp.int32) + lanes

            def scan_body(i, ptr):
                off = i * _L
                idx16 = idx_t[pl.ds(off, _L)]
                rel = idx16 - base16
                m = (rel >= zero16) & (rel < cells16)
                mi = m.astype(jnp.int32)
                incl = plsc.cumsum(mi)
                pos = incl - mi + jnp.full((_L,), ptr, jnp.int32)
                pos = jnp.where(m, pos, trash16)
                vox = vox0 + jnp.full((_L,), off, jnp.int32)
                plsc.store_scatter(pend_vox, [pos], jnp.where(m, vox, zero16))
                plsc.store_scatter(pend_cell, [pos],
                                   jnp.where(m, rel, dump16))
                return ptr + jnp.max(incl)

            ptr = lax.fori_loop(0, nvec, scan_body, 0)

            for t in range(_K // _L):
                pp = jnp.full((_L,), ptr + t * _L, jnp.int32) + lanes
                plsc.store_scatter(pend_vox, [pp], zero16)
                plsc.store_scatter(pend_cell, [pp], dump16)

            nch = (ptr + _K - 1) // _K

            def fire_body(j, carry):
                off = j * _K
                vox16 = pend_vox[pl.ds(off, _L)]
                cell16 = pend_cell[pl.ds(off, _L)]
                pltpu.async_copy(feat_h.at[vox16], rows, gsem).wait()
                pltpu.sync_copy(rows, acc.at[cell16], add=True)
                pltpu.sync_copy(ones, cnt.at[cell16], add=True)
                return carry

            lax.fori_loop(0, nch, fire_body, 0)
            plsc.subcore_barrier()

            g0 = base + row0
            pltpu.sync_copy(acc.at[pl.ds(row0, half)], sums_h.at[pl.ds(g0, half)])
            pltpu.sync_copy(acc.at[pl.ds(row0 + half, half)],
                            sums_h.at[pl.ds(g0 + half, half)])
            pltpu.sync_copy(cnt.at[pl.ds(row0, _RPT)], cnts_h.at[pl.ds(g0, _RPT)])
            pltpu.sync_copy(zrows, acc.at[pl.ds(row0, half)])
            pltpu.sync_copy(zrows, acc.at[pl.ds(row0 + half, half)])
            pltpu.sync_copy(zcnt, cnt.at[pl.ds(row0, _RPT)])
            plsc.subcore_barrier()
            return carry

        lax.fori_loop(0, _NPASS, pass_body, 0)

    return k(feat, flat_idx)


def _tc_project_ln(sums, cnts, w_proj, b_proj, gamma, beta):
    bev = w_proj.shape[0]
    rb = 1024
    grid = (_NCELL // rb,)

    def body(a_ref, w_ref, b_ref, g_ref, bt_ref, c_ref, o_ref):
        a = a_ref[...]
        x = lax.dot_general(a, w_ref[...], (((1,), (1,)), ((), ())),
                            preferred_element_type=jnp.float32)
        x = x + c_ref[:, 0:1] * b_ref[...]
        mu = jnp.mean(x, axis=1, keepdims=True)
        xc = x - mu
        var = jnp.mean(xc * xc, axis=1, keepdims=True)
        y = xc * lax.rsqrt(var + 1e-5)
        o_ref[...] = y * g_ref[...] + bt_ref[...]

    return pl.pallas_call(
        body,
        grid=grid,
        in_specs=[
            pl.BlockSpec((rb, _D), lambda i: (i, 0)),
            pl.BlockSpec((bev, _D), lambda i: (0, 0)),
            pl.BlockSpec((1, bev), lambda i: (0, 0)),
            pl.BlockSpec((1, bev), lambda i: (0, 0)),
            pl.BlockSpec((1, bev), lambda i: (0, 0)),
            pl.BlockSpec((rb, _L), lambda i: (i, 0)),
        ],
        out_specs=pl.BlockSpec((rb, bev), lambda i: (i, 0)),
        out_shape=jax.ShapeDtypeStruct((_NCELL, bev), jnp.float32),
        compiler_params=pltpu.CompilerParams(
            dimension_semantics=("arbitrary",)),
    )(sums, w_proj, b_proj, gamma, beta, cnts)


def kernel(voxel_features, coords, batch_size, grid_h, grid_w, W_proj,
           b_proj, gamma, beta):
    flat = (coords[:, 0] * (_H * _W) + coords[:, 1] * _W
            + coords[:, 2]).astype(jnp.int32)
    sums, cnts = _sc_segment_sum(voxel_features, flat)
    bev = W_proj.shape[0]
    out = _tc_project_ln(sums, cnts, W_proj, b_proj.reshape(1, bev),
                         gamma.reshape(1, bev), beta.reshape(1, bev))
    return out.reshape(_B, _H, _W, bev)
